# Initial kernel scaffold; baseline (speedup 1.0000x reference)
import jax
import jax.numpy as jnp
from jax.experimental import pallas as pl


def kernel(edge_index, emb, W1, b1, W2, b2, gamma, beta):
    raise NotImplementedError("write your pallas kernel here")


# speedup vs baseline: 9.4725x; 9.4725x over previous
import functools

import jax
import jax.numpy as jnp
from jax import lax
from jax.experimental import pallas as pl
from jax.experimental.pallas import tpu as pltpu
from jax.experimental.pallas import tpu_sc as plsc

N = 10000
D = 128
E = 320000

NC = 2
NS = 16
NW = NC * NS
CH = 128
NCHUNK = 157
EPT = CH * NCHUNK
EP = EPT * NS
NCHUNK2 = 79
EPT2 = CH * NCHUNK2
NP = N + 16

_MESH = plsc.VectorSubcoreMesh(core_axis_name="c", subcore_axis_name="s")


def _per_tile_rows(sid, total, fn):
    base = -(-((total + NS - 1) // NS) // 8) * 8
    last = total - (NS - 1) * base
    row0 = sid * base

    @pl.when(sid < NS - 1)
    def _():
        fn(row0, base)

    @pl.when(sid == NS - 1)
    def _():
        fn(row0, last)


def _deg_body(dst_hbm, zeros_hbm, ones_hbm, out_hbm, idx_d, ones_v, stage_v,
              acc_s):
    cid = lax.axis_index("c")
    sid = lax.axis_index("s")
    pltpu.sync_copy(zeros_hbm, stage_v)
    pltpu.sync_copy(ones_hbm, ones_v)

    def zero(row0, sz):
        for off in range(0, sz, CH):
            c = min(CH, sz - off)
            pltpu.sync_copy(stage_v.at[pl.ds(0, c)],
                            acc_s.at[pl.ds(row0 + off, c)])

    _per_tile_rows(sid, NP, zero)
    plsc.subcore_barrier()

    def chunk(i, _):
        pltpu.sync_copy(dst_hbm.at[pl.ds((sid * NCHUNK + i) * CH, CH)],
                        idx_d)
        pltpu.sync_copy(ones_v, acc_s.at[idx_d], add=True)
        return 0

    lax.fori_loop(0, NCHUNK, chunk, 0)
    plsc.subcore_barrier()

    def writeout(row0, sz):
        for off in range(0, sz, CH):
            c = min(CH, sz - off)
            pltpu.sync_copy(acc_s.at[pl.ds(row0 + off, c)],
                            stage_v.at[pl.ds(0, c)])
            pltpu.sync_copy(stage_v.at[pl.ds(0, c)],
                            out_hbm.at[pl.ds(cid * N + row0 + off, c)])

    _per_tile_rows(sid, N, writeout)


_deg_kernel = functools.partial(
    pl.kernel,
    out_type=jax.ShapeDtypeStruct((2 * N, D), jnp.float32),
    mesh=_MESH,
    scratch_types=[
        pltpu.VMEM((CH,), jnp.int32),
        pltpu.VMEM((CH, D), jnp.float32),
        pltpu.VMEM((CH, D), jnp.float32),
        pltpu.VMEM_SHARED((NP, D), jnp.float32),
    ],
)(_deg_body)


def _make_edge_scatter(FH, NCH):
    def body(xs_hbm, src_hbm, dst_hbm, y_hbm, idx_s, idx_d, rows_v, acc_s, sem):
        cid = lax.axis_index("c")
        sid = lax.axis_index("s")

        def seed(row0, sz):
            for off in range(0, sz, CH):
                c = min(CH, sz - off)
                pltpu.sync_copy(xs_hbm.at[pl.ds(cid * N + row0 + off, c)],
                                rows_v.at[pl.ds(0, c)])
                pltpu.sync_copy(rows_v.at[pl.ds(0, c)],
                                acc_s.at[pl.ds(row0 + off, c)])

        _per_tile_rows(sid, N, seed)
        plsc.subcore_barrier()

        ibase = (cid * NS + sid) * NCH * CH

        def chunk(i, _):
            pltpu.sync_copy(src_hbm.at[pl.ds(ibase + i * CH, CH)], idx_s)
            pltpu.sync_copy(dst_hbm.at[pl.ds(ibase + i * CH, CH)], idx_d)
            pltpu.async_copy(xs_hbm.at[idx_s], rows_v, sem).wait()
            pltpu.sync_copy(rows_v, acc_s.at[idx_d], add=True)
            return 0

        lax.fori_loop(0, NCH, chunk, 0)
        plsc.subcore_barrier()

        def writeout(row0, sz):
            for off in range(0, sz, CH):
                c = min(CH, sz - off)
                pltpu.sync_copy(acc_s.at[pl.ds(row0 + off, c)],
                                rows_v.at[pl.ds(0, c)])
                pltpu.sync_copy(rows_v.at[pl.ds(0, c)],
                                y_hbm.at[pl.ds(cid * N + row0 + off, c)])

        _per_tile_rows(sid, N, writeout)

    return functools.partial(
        pl.kernel,
        out_type=jax.ShapeDtypeStruct((2 * N, FH), jnp.float32),
        mesh=_MESH,
        scratch_types=[
            pltpu.VMEM((CH,), jnp.int32),
            pltpu.VMEM((CH,), jnp.int32),
            pltpu.VMEM((CH, FH), jnp.float32),
            pltpu.VMEM_SHARED((NP, FH), jnp.float32),
            pltpu.SemaphoreType.DMA,
        ],
    )(body)


_edge_scatter_l1 = _make_edge_scatter(128, NCHUNK)
_edge_scatter_l2 = _make_edge_scatter(128, NCHUNK2)

_BN = 1000
_GB = N // _BN


def _mm1_body(deg_ref, emb_ref, w1_ref, dinv_ref, xs_ref):
    deg = 1.0 + deg_ref[...][:, :1]
    dinv = lax.rsqrt(deg)
    dinv_ref[...] = dinv
    xw = jnp.dot(emb_ref[...], w1_ref[...],
                 preferred_element_type=jnp.float32)
    xs_ref[...] = xw * dinv


def _mm1(deg16, emb, W1):
    return pl.pallas_call(
        _mm1_body,
        grid=(_GB, 2),
        in_specs=[
            pl.BlockSpec((_BN, D), lambda i, j: (i, 0)),
            pl.BlockSpec((_BN, D), lambda i, j: (i, 0)),
            pl.BlockSpec((D, D), lambda i, j: (0, j)),
        ],
        out_specs=[
            pl.BlockSpec((_BN, 1), lambda i, j: (i, 0)),
            pl.BlockSpec((_BN, D), lambda i, j: (j * _GB + i, 0)),
        ],
        out_shape=[
            jax.ShapeDtypeStruct((N, 1), jnp.float32),
            jax.ShapeDtypeStruct((2 * N, D), jnp.float32),
        ],
    )(deg16, emb, W1)


def _mm2_body(y1a_ref, y1b_ref, dinv_ref, b1_ref, w2_ref, xs2_ref):
    dinv = dinv_ref[...]
    b1 = b1_ref[...]
    ha = y1a_ref[...] * dinv + b1[:, :D]
    hb = y1b_ref[...] * dinv + b1[:, D:]
    w2 = w2_ref[...]
    xw = (jnp.dot(ha, w2[:D], preferred_element_type=jnp.float32)
          + jnp.dot(hb, w2[D:], preferred_element_type=jnp.float32))
    xs2_ref[...] = xw * dinv


def _mm2(y1, dinv, b1, W2):
    return pl.pallas_call(
        _mm2_body,
        grid=(_GB,),
        in_specs=[
            pl.BlockSpec((_BN, D), lambda i: (i, 0)),
            pl.BlockSpec((_BN, D), lambda i: (i + _GB, 0)),
            pl.BlockSpec((_BN, 1), lambda i: (i, 0)),
            pl.BlockSpec((1, 2 * D), lambda i: (0, 0)),
            pl.BlockSpec((2 * D, D), lambda i: (0, 0)),
        ],
        out_specs=pl.BlockSpec((_BN, D), lambda i: (i, 0)),
        out_shape=jax.ShapeDtypeStruct((N, D), jnp.float32),
    )(y1, y1, dinv, b1.reshape(1, 2 * D), W2)


def _bn_body(y2_ref, dinv_ref, b2_ref, g_ref, bt_ref, out_ref):
    y2 = y2_ref[...]
    z = (y2[:N] + y2[N:]) * dinv_ref[...] + b2_ref[...]
    mean = jnp.mean(z, axis=0, keepdims=True)
    zc = z - mean
    var = jnp.mean(zc * zc, axis=0, keepdims=True)
    out_ref[...] = zc * lax.rsqrt(var + 1e-5) * g_ref[...] + bt_ref[...]


def _bn(y2, dinv, b2, gamma, beta):
    return pl.pallas_call(
        _bn_body,
        out_shape=jax.ShapeDtypeStruct((N, D), jnp.float32),
    )(y2, dinv, b2.reshape(1, D), gamma.reshape(1, D), beta.reshape(1, D))


def kernel(edge_index, emb, W1, b1, W2, b2, gamma, beta):
    ei = edge_index.astype(jnp.int32)
    src, dst = ei[0], ei[1]
    pad = EP - E
    srcp = jnp.concatenate([src, jnp.zeros((pad,), jnp.int32)])
    dstp = jnp.concatenate([dst, jnp.full((pad,), N, jnp.int32)])
    src2 = jnp.concatenate([srcp, srcp + N])
    dst2 = jnp.concatenate([dstp, dstp])
    eh = E // NC
    ept = eh // NS
    ptile = EPT2 - ept
    src3 = jnp.pad(src.reshape(NC * NS, ept),
                   ((0, 0), (0, ptile))).reshape(-1)
    dst3 = jnp.pad(dst.reshape(NC * NS, ept), ((0, 0), (0, ptile)),
                   constant_values=N).reshape(-1)

    deg16 = _deg_kernel(dstp, jnp.zeros((CH, D), jnp.float32),
                        jnp.ones((CH, D), jnp.float32))
    dinv, xs1 = _mm1(deg16, emb, W1)
    y1 = _edge_scatter_l1(xs1, src2, dst2)
    xs2 = _mm2(y1, dinv, b1, W2)
    xs2p = jnp.concatenate([xs2, jnp.zeros((N, D), jnp.float32)])
    y2 = _edge_scatter_l2(xs2p, src3, dst3)
    return _bn(y2, dinv, b2, gamma, beta)


# speedup vs baseline: 10.0487x; 1.0608x over previous
import functools

import jax
import jax.numpy as jnp
from jax import lax
from jax.experimental import pallas as pl
from jax.experimental.pallas import tpu as pltpu
from jax.experimental.pallas import tpu_sc as plsc

N = 10000
D = 128
E = 320000

NC = 2
NS = 16
CH = 128
NCHUNK = 158
EPT = CH * NCHUNK
EP = EPT * NS
NCHUNK2 = 80
EPT2 = CH * NCHUNK2
NP = N + 16

_MESH = plsc.VectorSubcoreMesh(core_axis_name="c", subcore_axis_name="s")


def _per_tile_rows(sid, total, fn):
    base = -(-((total + NS - 1) // NS) // 8) * 8
    last = total - (NS - 1) * base
    row0 = sid * base

    @pl.when(sid < NS - 1)
    def _():
        fn(row0, base)

    @pl.when(sid == NS - 1)
    def _():
        fn(row0, last)


def _deg_body(dst_hbm, zeros_hbm, ones_hbm, out_hbm, idx_d, ones_v, stage_v,
              acc_s):
    cid = lax.axis_index("c")
    sid = lax.axis_index("s")
    pltpu.sync_copy(zeros_hbm, stage_v)
    pltpu.sync_copy(ones_hbm, ones_v)

    def zero(row0, sz):
        for off in range(0, sz, CH):
            c = min(CH, sz - off)
            pltpu.sync_copy(stage_v.at[pl.ds(0, c)],
                            acc_s.at[pl.ds(row0 + off, c)])

    _per_tile_rows(sid, NP, zero)
    plsc.subcore_barrier()
    ibase = (cid * NS + sid) * NCHUNK2 * CH

    def chunk(i, _):
        pltpu.sync_copy(dst_hbm.at[pl.ds(ibase + i * CH, CH)], idx_d)
        pltpu.sync_copy(ones_v, acc_s.at[idx_d], add=True)
        return 0

    lax.fori_loop(0, NCHUNK2, chunk, 0)
    plsc.subcore_barrier()

    def writeout(row0, sz):
        for off in range(0, sz, CH):
            c = min(CH, sz - off)
            pltpu.sync_copy(acc_s.at[pl.ds(row0 + off, c)],
                            stage_v.at[pl.ds(0, c)])
            pltpu.sync_copy(stage_v.at[pl.ds(0, c)],
                            out_hbm.at[pl.ds(cid * N + row0 + off, c)])

    _per_tile_rows(sid, N, writeout)


_deg_kernel = functools.partial(
    pl.kernel,
    out_type=jax.ShapeDtypeStruct((2 * N, D), jnp.float32),
    mesh=_MESH,
    scratch_types=[
        pltpu.VMEM((CH,), jnp.int32),
        pltpu.VMEM((CH, D), jnp.float32),
        pltpu.VMEM((CH, D), jnp.float32),
        pltpu.VMEM_SHARED((NP, D), jnp.float32),
    ],
)(_deg_body)


def _make_edge_scatter(NCH):
    P = NCH // 2
    assert NCH == 2 * P

    def body(xs_hbm, src_hbm, dst_hbm, y_hbm, s_a, d_a, s_b, d_b,
             rows_a, rows_b, acc_s, sem_a, sem_b):
        cid = lax.axis_index("c")
        sid = lax.axis_index("s")

        def seed(row0, sz):
            for off in range(0, sz, CH):
                c = min(CH, sz - off)
                pltpu.sync_copy(xs_hbm.at[pl.ds(cid * N + row0 + off, c)],
                                rows_a.at[pl.ds(0, c)])
                pltpu.sync_copy(rows_a.at[pl.ds(0, c)],
                                acc_s.at[pl.ds(row0 + off, c)])

        _per_tile_rows(sid, N, seed)
        plsc.subcore_barrier()

        ibase = (cid * NS + sid) * NCH * CH

        def load(i, s, d):
            pltpu.sync_copy(src_hbm.at[pl.ds(ibase + i * CH, CH)], s)
            pltpu.sync_copy(dst_hbm.at[pl.ds(ibase + i * CH, CH)], d)

        def start_a():
            pltpu.make_async_copy(xs_hbm.at[s_a], rows_a, sem_a).start()

        def wait_a():
            pltpu.make_async_copy(xs_hbm.at[s_a], rows_a, sem_a).wait()

        def start_b():
            pltpu.make_async_copy(xs_hbm.at[s_b], rows_b, sem_b).start()

        def wait_b():
            pltpu.make_async_copy(xs_hbm.at[s_b], rows_b, sem_b).wait()

        def scat_a():
            pltpu.sync_copy(rows_a, acc_s.at[d_a], add=True)

        def scat_b():
            pltpu.sync_copy(rows_b, acc_s.at[d_b], add=True)

        load(0, s_a, d_a)
        start_a()

        def pair(g, _):
            load(2 * g + 1, s_b, d_b)
            wait_a()
            start_b()
            scat_a()
            load(2 * g + 2, s_a, d_a)
            wait_b()
            start_a()
            scat_b()
            return 0

        lax.fori_loop(0, P - 1, pair, 0)
        load(NCH - 1, s_b, d_b)
        wait_a()
        start_b()
        scat_a()
        wait_b()
        scat_b()
        plsc.subcore_barrier()

        def writeout(row0, sz):
            for off in range(0, sz, CH):
                c = min(CH, sz - off)
                pltpu.sync_copy(acc_s.at[pl.ds(row0 + off, c)],
                                rows_a.at[pl.ds(0, c)])
                pltpu.sync_copy(rows_a.at[pl.ds(0, c)],
                                y_hbm.at[pl.ds(cid * N + row0 + off, c)])

        _per_tile_rows(sid, N, writeout)

    return functools.partial(
        pl.kernel,
        out_type=jax.ShapeDtypeStruct((2 * N, D), jnp.float32),
        mesh=_MESH,
        scratch_types=[
            pltpu.VMEM((CH,), jnp.int32),
            pltpu.VMEM((CH,), jnp.int32),
            pltpu.VMEM((CH,), jnp.int32),
            pltpu.VMEM((CH,), jnp.int32),
            pltpu.VMEM((CH, D), jnp.float32),
            pltpu.VMEM((CH, D), jnp.float32),
            pltpu.VMEM_SHARED((NP, D), jnp.float32),
            pltpu.SemaphoreType.DMA,
            pltpu.SemaphoreType.DMA,
        ],
    )(body)


_edge_scatter_l1 = _make_edge_scatter(NCHUNK)
_edge_scatter_l2 = _make_edge_scatter(NCHUNK2)

_BN = 1000
_GB = N // _BN


def _mm1_body(dega_ref, degb_ref, emb_ref, w1_ref, dinv_ref, xs_ref):
    deg = 1.0 + dega_ref[...][:, :1] + degb_ref[...][:, :1]
    dinv = lax.rsqrt(deg)
    dinv_ref[...] = dinv
    xw = jnp.dot(emb_ref[...], w1_ref[...],
                 preferred_element_type=jnp.float32)
    xs_ref[...] = xw * dinv


def _mm1(deg2, emb, W1):
    return pl.pallas_call(
        _mm1_body,
        grid=(_GB, 2),
        in_specs=[
            pl.BlockSpec((_BN, D), lambda i, j: (i, 0)),
            pl.BlockSpec((_BN, D), lambda i, j: (i + _GB, 0)),
            pl.BlockSpec((_BN, D), lambda i, j: (i, 0)),
            pl.BlockSpec((D, D), lambda i, j: (0, j)),
        ],
        out_specs=[
            pl.BlockSpec((_BN, 1), lambda i, j: (i, 0)),
            pl.BlockSpec((_BN, D), lambda i, j: (j * _GB + i, 0)),
        ],
        out_shape=[
            jax.ShapeDtypeStruct((N, 1), jnp.float32),
            jax.ShapeDtypeStruct((2 * N, D), jnp.float32),
        ],
    )(deg2, deg2, emb, W1)


def _mm2_body(y1a_ref, y1b_ref, dinv_ref, b1_ref, w2_ref, xs2_ref):
    dinv = dinv_ref[...]
    b1 = b1_ref[...]
    ha = y1a_ref[...] * dinv + b1[:, :D]
    hb = y1b_ref[...] * dinv + b1[:, D:]
    w2 = w2_ref[...]
    xw = (jnp.dot(ha, w2[:D], preferred_element_type=jnp.float32)
          + jnp.dot(hb, w2[D:], preferred_element_type=jnp.float32))
    xs2_ref[...] = xw * dinv


def _mm2(y1, dinv, b1, W2):
    return pl.pallas_call(
        _mm2_body,
        grid=(_GB,),
        in_specs=[
            pl.BlockSpec((_BN, D), lambda i: (i, 0)),
            pl.BlockSpec((_BN, D), lambda i: (i + _GB, 0)),
            pl.BlockSpec((_BN, 1), lambda i: (i, 0)),
            pl.BlockSpec((1, 2 * D), lambda i: (0, 0)),
            pl.BlockSpec((2 * D, D), lambda i: (0, 0)),
        ],
        out_specs=pl.BlockSpec((_BN, D), lambda i: (i, 0)),
        out_shape=jax.ShapeDtypeStruct((N, D), jnp.float32),
    )(y1, y1, dinv, b1.reshape(1, 2 * D), W2)


def _bn_body(y2_ref, dinv_ref, b2_ref, g_ref, bt_ref, out_ref):
    y2 = y2_ref[...]
    z = (y2[:N] + y2[N:]) * dinv_ref[...] + b2_ref[...]
    mean = jnp.mean(z, axis=0, keepdims=True)
    zc = z - mean
    var = jnp.mean(zc * zc, axis=0, keepdims=True)
    out_ref[...] = zc * lax.rsqrt(var + 1e-5) * g_ref[...] + bt_ref[...]


def _bn(y2, dinv, b2, gamma, beta):
    return pl.pallas_call(
        _bn_body,
        out_shape=jax.ShapeDtypeStruct((N, D), jnp.float32),
    )(y2, dinv, b2.reshape(1, D), gamma.reshape(1, D), beta.reshape(1, D))


def kernel(edge_index, emb, W1, b1, W2, b2, gamma, beta):
    ei = edge_index.astype(jnp.int32)
    src, dst = ei[0], ei[1]
    pad = EP - E
    srcp = jnp.concatenate([src, jnp.zeros((pad,), jnp.int32)])
    dstp = jnp.concatenate([dst, jnp.full((pad,), N, jnp.int32)])
    src2 = jnp.concatenate([srcp, srcp + N])
    dst2 = jnp.concatenate([dstp, dstp])
    eh = E // NC
    ept = eh // NS
    ptile = EPT2 - ept
    src3 = jnp.pad(src.reshape(NC * NS, ept),
                   ((0, 0), (0, ptile))).reshape(-1)
    dst3 = jnp.pad(dst.reshape(NC * NS, ept), ((0, 0), (0, ptile)),
                   constant_values=N).reshape(-1)

    deg2 = _deg_kernel(dst3, jnp.zeros((CH, D), jnp.float32),
                       jnp.ones((CH, D), jnp.float32))
    dinv, xs1 = _mm1(deg2, emb, W1)
    y1 = _edge_scatter_l1(xs1, src2, dst2)
    xs2 = _mm2(y1, dinv, b1, W2)
    xs2p = jnp.concatenate([xs2, jnp.zeros((N, D), jnp.float32)])
    y2 = _edge_scatter_l2(xs2p, src3, dst3)
    return _bn(y2, dinv, b2, gamma, beta)


# speedup vs baseline: 11.9183x; 1.1861x over previous
import functools

import jax
import jax.numpy as jnp
from jax import lax
from jax.experimental import pallas as pl
from jax.experimental.pallas import tpu as pltpu
from jax.experimental.pallas import tpu_sc as plsc

N = 10000
D = 128
E = 320000

NC = 2
NS = 16
CH = 128
NCHUNK = 158
EPT = CH * NCHUNK
EP = EPT * NS
NCHUNK2 = 80
EPT2 = CH * NCHUNK2
NP = N + 16

_MESH = plsc.VectorSubcoreMesh(core_axis_name="c", subcore_axis_name="s")


def _per_tile_rows(sid, total, fn):
    base = -(-((total + NS - 1) // NS) // 8) * 8
    last = total - (NS - 1) * base
    row0 = sid * base

    @pl.when(sid < NS - 1)
    def _():
        fn(row0, base)

    @pl.when(sid == NS - 1)
    def _():
        fn(row0, last)


def _deg_body(dst_hbm, zeros_hbm, ones_hbm, out_hbm, d_a, d_b, ones_v,
              stage_v, acc_s, sem_a, sem_b):
    cid = lax.axis_index("c")
    sid = lax.axis_index("s")
    pltpu.sync_copy(zeros_hbm, stage_v)
    pltpu.sync_copy(ones_hbm, ones_v)

    def zero(row0, sz):
        for off in range(0, sz, CH):
            c = min(CH, sz - off)
            pltpu.sync_copy(stage_v.at[pl.ds(0, c)],
                            acc_s.at[pl.ds(row0 + off, c)])

    _per_tile_rows(sid, NP, zero)
    plsc.subcore_barrier()
    ibase = (cid * NS + sid) * NCHUNK2 * CH

    def start_l(i, d, sem):
        pltpu.make_async_copy(dst_hbm.at[pl.ds(ibase + i * CH, CH)], d,
                              sem).start()

    def wait_l(i, d, sem):
        pltpu.make_async_copy(dst_hbm.at[pl.ds(ibase + i * CH, CH)], d,
                              sem).wait()

    P2 = NCHUNK2 // 2
    start_l(0, d_a, sem_a)

    def pair(g, _):
        start_l(2 * g + 1, d_b, sem_b)
        wait_l(2 * g, d_a, sem_a)
        pltpu.sync_copy(ones_v, acc_s.at[d_a], add=True)
        start_l(2 * g + 2, d_a, sem_a)
        wait_l(2 * g + 1, d_b, sem_b)
        pltpu.sync_copy(ones_v, acc_s.at[d_b], add=True)
        return 0

    lax.fori_loop(0, P2 - 1, pair, 0)
    start_l(NCHUNK2 - 1, d_b, sem_b)
    wait_l(NCHUNK2 - 2, d_a, sem_a)
    pltpu.sync_copy(ones_v, acc_s.at[d_a], add=True)
    wait_l(NCHUNK2 - 1, d_b, sem_b)
    pltpu.sync_copy(ones_v, acc_s.at[d_b], add=True)
    plsc.subcore_barrier()

    def writeout(row0, sz):
        for off in range(0, sz, CH):
            c = min(CH, sz - off)
            pltpu.sync_copy(acc_s.at[pl.ds(row0 + off, c)],
                            stage_v.at[pl.ds(0, c)])
            pltpu.sync_copy(stage_v.at[pl.ds(0, c)],
                            out_hbm.at[pl.ds(cid * N + row0 + off, c)])

    _per_tile_rows(sid, N, writeout)


_deg_kernel = functools.partial(
    pl.kernel,
    out_type=jax.ShapeDtypeStruct((2 * N, D), jnp.float32),
    mesh=_MESH,
    scratch_types=[
        pltpu.VMEM((CH,), jnp.int32),
        pltpu.VMEM((CH,), jnp.int32),
        pltpu.VMEM((CH, D), jnp.float32),
        pltpu.VMEM((CH, D), jnp.float32),
        pltpu.VMEM_SHARED((NP, D), jnp.float32),
        pltpu.SemaphoreType.DMA,
        pltpu.SemaphoreType.DMA,
    ],
)(_deg_body)


def _make_edge_scatter(NCH):
    P = NCH // 2
    assert NCH == 2 * P

    def body(xs_hbm, src_hbm, dst_hbm, y_hbm, s_a, d_a, s_b, d_b,
             rows_a, rows_b, acc_s, sem_a, sem_b):
        cid = lax.axis_index("c")
        sid = lax.axis_index("s")
        nreg = xs_hbm.shape[0] // (2 * N)

        def seed(row0, sz):
            sbase = cid * N * (2 if nreg > 1 else 1)
            for off in range(0, sz, CH):
                c = min(CH, sz - off)
                pltpu.sync_copy(xs_hbm.at[pl.ds(sbase + row0 + off, c)],
                                rows_a.at[pl.ds(0, c)])
                pltpu.sync_copy(rows_a.at[pl.ds(0, c)],
                                acc_s.at[pl.ds(row0 + off, c)])

        _per_tile_rows(sid, N, seed)
        plsc.subcore_barrier()

        ibase = (cid * NS + sid) * NCH * CH
        nd = dst_hbm.shape[0]
        dbase = ibase % nd

        def load(i, s, d):
            pltpu.sync_copy(src_hbm.at[pl.ds(ibase + i * CH, CH)], s)
            pltpu.sync_copy(dst_hbm.at[pl.ds(dbase + i * CH, CH)], d)

        def start_a():
            pltpu.make_async_copy(xs_hbm.at[s_a], rows_a, sem_a).start()

        def wait_a():
            pltpu.make_async_copy(xs_hbm.at[s_a], rows_a, sem_a).wait()

        def start_b():
            pltpu.make_async_copy(xs_hbm.at[s_b], rows_b, sem_b).start()

        def wait_b():
            pltpu.make_async_copy(xs_hbm.at[s_b], rows_b, sem_b).wait()

        def scat_a():
            pltpu.sync_copy(rows_a, acc_s.at[d_a], add=True)

        def scat_b():
            pltpu.sync_copy(rows_b, acc_s.at[d_b], add=True)

        load(0, s_a, d_a)
        start_a()

        def pair(g, _):
            load(2 * g + 1, s_b, d_b)
            wait_a()
            start_b()
            scat_a()
            load(2 * g + 2, s_a, d_a)
            wait_b()
            start_a()
            scat_b()
            return 0

        lax.fori_loop(0, P - 1, pair, 0)
        load(NCH - 1, s_b, d_b)
        wait_a()
        start_b()
        scat_a()
        wait_b()
        scat_b()
        plsc.subcore_barrier()

        def writeout(row0, sz):
            for off in range(0, sz, CH):
                c = min(CH, sz - off)
                pltpu.sync_copy(acc_s.at[pl.ds(row0 + off, c)],
                                rows_a.at[pl.ds(0, c)])
                pltpu.sync_copy(rows_a.at[pl.ds(0, c)],
                                y_hbm.at[pl.ds(cid * N + row0 + off, c)])

        _per_tile_rows(sid, N, writeout)

    def make(xs, srcf, dstf):
        return functools.partial(
            pl.kernel,
            out_type=jax.ShapeDtypeStruct((2 * N, D), jnp.float32),
            mesh=_MESH,
            scratch_types=[
                pltpu.VMEM((CH,), jnp.int32),
                pltpu.VMEM((CH,), jnp.int32),
                pltpu.VMEM((CH,), jnp.int32),
                pltpu.VMEM((CH,), jnp.int32),
                pltpu.VMEM((CH, D), jnp.float32),
                pltpu.VMEM((CH, D), jnp.float32),
                pltpu.VMEM_SHARED((NP, D), jnp.float32),
                pltpu.SemaphoreType.DMA,
                pltpu.SemaphoreType.DMA,
            ],
        )(body)(xs, srcf, dstf)

    return make


_edge_scatter_l1 = _make_edge_scatter(NCHUNK)
_edge_scatter_l2 = _make_edge_scatter(NCHUNK2)

_BN = 1000
_GB = N // _BN


def _xw_body(emb_ref, w1_ref, xw_ref):
    xw_ref[...] = jnp.dot(emb_ref[...], w1_ref[...],
                          preferred_element_type=jnp.float32)


def _xw(emb, W1):
    return pl.pallas_call(
        _xw_body,
        grid=(_GB, 2),
        in_specs=[
            pl.BlockSpec((_BN, D), lambda i, j: (i, 0)),
            pl.BlockSpec((D, D), lambda i, j: (0, j)),
        ],
        out_specs=pl.BlockSpec((_BN, D), lambda i, j: (j * _GB + i, 0)),
        out_shape=jax.ShapeDtypeStruct((2 * N, D), jnp.float32),
    )(emb, W1)


def _scale_body(dega_ref, degb_ref, xw_ref, dinv_ref, xs_ref):
    deg = 1.0 + dega_ref[...][:, :1] + degb_ref[...][:, :1]
    dinv = lax.rsqrt(deg)
    dinv_ref[...] = dinv
    xs_ref[...] = xw_ref[...] * dinv


def _mm1(deg2, xw):
    return pl.pallas_call(
        _scale_body,
        grid=(_GB, 2),
        in_specs=[
            pl.BlockSpec((_BN, D), lambda i, j: (i, 0)),
            pl.BlockSpec((_BN, D), lambda i, j: (i + _GB, 0)),
            pl.BlockSpec((_BN, D), lambda i, j: (j * _GB + i, 0)),
        ],
        out_specs=[
            pl.BlockSpec((_BN, 1), lambda i, j: (i, 0)),
            pl.BlockSpec((_BN, D), lambda i, j: (j * _GB + i, 0)),
        ],
        out_shape=[
            jax.ShapeDtypeStruct((N, 1), jnp.float32),
            jax.ShapeDtypeStruct((2 * N, D), jnp.float32),
        ],
    )(deg2, deg2, xw)


def _mm2_body(y1a_ref, y1b_ref, dinv_ref, b1_ref, w2_ref, xs2_ref):
    j = pl.program_id(1)
    dinv = dinv_ref[...]
    b1 = b1_ref[...]
    ha = y1a_ref[...] * dinv + b1[:, :D]
    hb = y1b_ref[...] * dinv + b1[:, D:]
    w2 = w2_ref[...]
    xw = (jnp.dot(ha, w2[:D], preferred_element_type=jnp.float32)
          + jnp.dot(hb, w2[D:], preferred_element_type=jnp.float32))
    xs2_ref[...] = jnp.where(j < 2, xw * dinv, 0.0)


def _mm2(y1, dinv, b1, W2):
    return pl.pallas_call(
        _mm2_body,
        grid=(_GB, 4),
        in_specs=[
            pl.BlockSpec((_BN, D), lambda i, j: (i, 0)),
            pl.BlockSpec((_BN, D), lambda i, j: (i + _GB, 0)),
            pl.BlockSpec((_BN, 1), lambda i, j: (i, 0)),
            pl.BlockSpec((1, 2 * D), lambda i, j: (0, 0)),
            pl.BlockSpec((2 * D, D), lambda i, j: (0, 0)),
        ],
        out_specs=pl.BlockSpec((_BN, D), lambda i, j: (j * _GB + i, 0)),
        out_shape=jax.ShapeDtypeStruct((4 * N, D), jnp.float32),
    )(y1, y1, dinv, b1.reshape(1, 2 * D), W2)


def _bn_body(y2_ref, dinv_ref, b2_ref, g_ref, bt_ref, out_ref):
    y2 = y2_ref[...]
    z = (y2[:N] + y2[N:]) * dinv_ref[...] + b2_ref[...]
    mean = jnp.mean(z, axis=0, keepdims=True)
    zc = z - mean
    var = jnp.mean(zc * zc, axis=0, keepdims=True)
    out_ref[...] = zc * lax.rsqrt(var + 1e-5) * g_ref[...] + bt_ref[...]


def _bn(y2, dinv, b2, gamma, beta):
    return pl.pallas_call(
        _bn_body,
        out_shape=jax.ShapeDtypeStruct((N, D), jnp.float32),
    )(y2, dinv, b2.reshape(1, D), gamma.reshape(1, D), beta.reshape(1, D))


def kernel(edge_index, emb, W1, b1, W2, b2, gamma, beta):
    ei = edge_index.astype(jnp.int32)
    src, dst = ei[0], ei[1]
    pad = EP - E
    srcp = jnp.concatenate([src, jnp.zeros((pad,), jnp.int32)])
    dstp = jnp.concatenate([dst, jnp.full((pad,), N, jnp.int32)])
    src2 = jnp.concatenate([srcp, srcp + N])
    eh = E // NC
    ept = eh // NS
    ptile = EPT2 - ept
    src3h = jnp.pad(src.reshape(NC, NS, ept), ((0, 0), (0, 0), (0, ptile)))
    src3 = jnp.concatenate([src3h[0].reshape(-1), src3h[1].reshape(-1) + N])
    dst3 = jnp.pad(dst.reshape(NC * NS, ept), ((0, 0), (0, ptile)),
                   constant_values=N).reshape(-1)

    xw = _xw(emb, W1)
    deg2 = _deg_kernel(dst3, jnp.zeros((CH, D), jnp.float32),
                       jnp.ones((CH, D), jnp.float32))
    dinv, xs1 = _mm1(deg2, xw)
    y1 = _edge_scatter_l1(xs1, src2, dstp)
    xs2p = _mm2(y1, dinv, b1, W2)
    y2 = _edge_scatter_l2(xs2p, src3, dst3)
    return _bn(y2, dinv, b2, gamma, beta)


# speedup vs baseline: 12.3852x; 1.0392x over previous
import functools

import jax
import jax.numpy as jnp
from jax import lax
from jax.experimental import pallas as pl
from jax.experimental.pallas import tpu as pltpu
from jax.experimental.pallas import tpu_sc as plsc

N = 10000
D = 128
E = 320000

NC = 2
NS = 16
CH = 128
NCHUNK = 158
EPT = CH * NCHUNK
EP = EPT * NS
NCHUNK2 = 80
EPT2 = CH * NCHUNK2
NP = N + 16

_MESH = plsc.VectorSubcoreMesh(core_axis_name="c", subcore_axis_name="s")


def _per_tile_rows(sid, total, fn):
    base = -(-((total + NS - 1) // NS) // 8) * 8
    last = total - (NS - 1) * base
    row0 = sid * base

    @pl.when(sid < NS - 1)
    def _():
        fn(row0, base)

    @pl.when(sid == NS - 1)
    def _():
        fn(row0, last)


def _deg_body(dst_hbm, zeros_hbm, ones_hbm, out_hbm, d_a, d_b, ones_v,
              stage_v, acc_s, sem_a, sem_b):
    cid = lax.axis_index("c")
    sid = lax.axis_index("s")
    pltpu.sync_copy(zeros_hbm, stage_v)
    pltpu.sync_copy(ones_hbm, ones_v)

    def zero(row0, sz):
        for off in range(0, sz, CH):
            c = min(CH, sz - off)
            pltpu.sync_copy(stage_v.at[pl.ds(0, c)],
                            acc_s.at[pl.ds(row0 + off, c)])

    _per_tile_rows(sid, NP, zero)
    plsc.subcore_barrier()
    ibase = (cid * NS + sid) * NCHUNK2 * CH

    def start_l(i, d, sem):
        pltpu.make_async_copy(dst_hbm.at[pl.ds(ibase + i * CH, CH)], d,
                              sem).start()

    def wait_l(i, d, sem):
        pltpu.make_async_copy(dst_hbm.at[pl.ds(ibase + i * CH, CH)], d,
                              sem).wait()

    P2 = NCHUNK2 // 2
    start_l(0, d_a, sem_a)

    def pair(g, _):
        start_l(2 * g + 1, d_b, sem_b)
        wait_l(2 * g, d_a, sem_a)
        pltpu.sync_copy(ones_v, acc_s.at[d_a], add=True)
        start_l(2 * g + 2, d_a, sem_a)
        wait_l(2 * g + 1, d_b, sem_b)
        pltpu.sync_copy(ones_v, acc_s.at[d_b], add=True)
        return 0

    lax.fori_loop(0, P2 - 1, pair, 0)
    start_l(NCHUNK2 - 1, d_b, sem_b)
    wait_l(NCHUNK2 - 2, d_a, sem_a)
    pltpu.sync_copy(ones_v, acc_s.at[d_a], add=True)
    wait_l(NCHUNK2 - 1, d_b, sem_b)
    pltpu.sync_copy(ones_v, acc_s.at[d_b], add=True)
    plsc.subcore_barrier()

    def writeout(row0, sz):
        pltpu.sync_copy(acc_s.at[pl.ds(row0, sz)],
                        out_hbm.at[pl.ds(cid * N + row0, sz)])

    _per_tile_rows(sid, N, writeout)


_deg_kernel = functools.partial(
    pl.kernel,
    out_type=jax.ShapeDtypeStruct((2 * N, D), jnp.float32),
    mesh=_MESH,
    scratch_types=[
        pltpu.VMEM((CH,), jnp.int32),
        pltpu.VMEM((CH,), jnp.int32),
        pltpu.VMEM((CH, D), jnp.float32),
        pltpu.VMEM((CH, D), jnp.float32),
        pltpu.VMEM_SHARED((NP, D), jnp.float32),
        pltpu.SemaphoreType.DMA,
        pltpu.SemaphoreType.DMA,
    ],
)(_deg_body)


def _make_edge_scatter(NCH):
    P = NCH // 2
    assert NCH == 2 * P

    def body(xs_hbm, src_hbm, dst_hbm, y_hbm, s_a, d_a, s_b, d_b,
             rows_a, rows_b, acc_s, sem_a, sem_b):
        cid = lax.axis_index("c")
        sid = lax.axis_index("s")
        nreg = xs_hbm.shape[0] // (2 * N)

        def seed(row0, sz):
            sbase = cid * N * (2 if nreg > 1 else 1)
            pltpu.sync_copy(xs_hbm.at[pl.ds(sbase + row0, sz)],
                            acc_s.at[pl.ds(row0, sz)])

        _per_tile_rows(sid, N, seed)
        plsc.subcore_barrier()

        ibase = (cid * NS + sid) * NCH * CH

        def load(i, s, d):
            pltpu.sync_copy(src_hbm.at[pl.ds(ibase + i * CH, CH)], s)
            pltpu.sync_copy(dst_hbm.at[pl.ds(ibase + i * CH, CH)], d)

        def start_a():
            pltpu.make_async_copy(xs_hbm.at[s_a], rows_a, sem_a).start()

        def wait_a():
            pltpu.make_async_copy(xs_hbm.at[s_a], rows_a, sem_a).wait()

        def start_b():
            pltpu.make_async_copy(xs_hbm.at[s_b], rows_b, sem_b).start()

        def wait_b():
            pltpu.make_async_copy(xs_hbm.at[s_b], rows_b, sem_b).wait()

        def scat_a():
            pltpu.sync_copy(rows_a, acc_s.at[d_a], add=True)

        def scat_b():
            pltpu.sync_copy(rows_b, acc_s.at[d_b], add=True)

        load(0, s_a, d_a)
        start_a()

        def pair(g, _):
            load(2 * g + 1, s_b, d_b)
            wait_a()
            start_b()
            scat_a()
            load(2 * g + 2, s_a, d_a)
            wait_b()
            start_a()
            scat_b()
            return 0

        lax.fori_loop(0, P - 1, pair, 0)
        load(NCH - 1, s_b, d_b)
        wait_a()
        start_b()
        scat_a()
        wait_b()
        scat_b()
        plsc.subcore_barrier()

        def writeout(row0, sz):
            pltpu.sync_copy(acc_s.at[pl.ds(row0, sz)],
                            y_hbm.at[pl.ds(cid * N + row0, sz)])

        _per_tile_rows(sid, N, writeout)

    def make(xs, srcf, dstf):
        return functools.partial(
            pl.kernel,
            out_type=jax.ShapeDtypeStruct((2 * N, D), jnp.float32),
            mesh=_MESH,
            scratch_types=[
                pltpu.VMEM((CH,), jnp.int32),
                pltpu.VMEM((CH,), jnp.int32),
                pltpu.VMEM((CH,), jnp.int32),
                pltpu.VMEM((CH,), jnp.int32),
                pltpu.VMEM((CH, D), jnp.float32),
                pltpu.VMEM((CH, D), jnp.float32),
                pltpu.VMEM_SHARED((NP, D), jnp.float32),
                pltpu.SemaphoreType.DMA,
                pltpu.SemaphoreType.DMA,
            ],
        )(body)(xs, srcf, dstf)

    return make


_edge_scatter_l1 = _make_edge_scatter(NCHUNK)
_edge_scatter_l2 = _make_edge_scatter(NCHUNK2)

_BN = 1000
_GB = N // _BN


def _xw_body(emb_ref, w1_ref, xw_ref):
    xw_ref[...] = jnp.dot(emb_ref[...], w1_ref[...],
                          preferred_element_type=jnp.float32)


def _xw(emb, W1):
    return pl.pallas_call(
        _xw_body,
        grid=(_GB, 2),
        in_specs=[
            pl.BlockSpec((_BN, D), lambda i, j: (i, 0)),
            pl.BlockSpec((D, D), lambda i, j: (0, j)),
        ],
        out_specs=pl.BlockSpec((_BN, D), lambda i, j: (j * _GB + i, 0)),
        out_shape=jax.ShapeDtypeStruct((2 * N, D), jnp.float32),
    )(emb, W1)


def _scale_body(dega_ref, degb_ref, xw_ref, dinv_ref, xs_ref):
    deg = 1.0 + dega_ref[...][:, :1] + degb_ref[...][:, :1]
    dinv = lax.rsqrt(deg)
    dinv_ref[...] = dinv
    xs_ref[...] = xw_ref[...] * dinv


def _mm1(deg2, xw):
    return pl.pallas_call(
        _scale_body,
        grid=(_GB, 2),
        in_specs=[
            pl.BlockSpec((_BN, D), lambda i, j: (i, 0)),
            pl.BlockSpec((_BN, D), lambda i, j: (i + _GB, 0)),
            pl.BlockSpec((_BN, D), lambda i, j: (j * _GB + i, 0)),
        ],
        out_specs=[
            pl.BlockSpec((_BN, 1), lambda i, j: (i, 0)),
            pl.BlockSpec((_BN, D), lambda i, j: (j * _GB + i, 0)),
        ],
        out_shape=[
            jax.ShapeDtypeStruct((N, 1), jnp.float32),
            jax.ShapeDtypeStruct((2 * N, D), jnp.float32),
        ],
    )(deg2, deg2, xw)


def _mm2_body(y1a_ref, y1b_ref, dinv_ref, b1_ref, w2_ref, xs2_ref):
    dinv = dinv_ref[...]
    b1 = b1_ref[...]
    ha = y1a_ref[...] * dinv + b1[:, :D]
    hb = y1b_ref[...] * dinv + b1[:, D:]
    w2 = w2_ref[...]
    xw = (jnp.dot(ha, w2[:D], preferred_element_type=jnp.float32)
          + jnp.dot(hb, w2[D:], preferred_element_type=jnp.float32))
    xs2_ref[...] = xw * dinv


def _mm2(y1, dinv, b1, W2):
    return pl.pallas_call(
        _mm2_body,
        grid=(_GB,),
        in_specs=[
            pl.BlockSpec((_BN, D), lambda i: (i, 0)),
            pl.BlockSpec((_BN, D), lambda i: (i + _GB, 0)),
            pl.BlockSpec((_BN, 1), lambda i: (i, 0)),
            pl.BlockSpec((1, 2 * D), lambda i: (0, 0)),
            pl.BlockSpec((2 * D, D), lambda i: (0, 0)),
        ],
        out_specs=pl.BlockSpec((_BN, D), lambda i: (i, 0)),
        out_shape=jax.ShapeDtypeStruct((N, D), jnp.float32),
    )(y1, y1, dinv, b1.reshape(1, 2 * D), W2)


def _bn_body(y2_ref, dinv_ref, b2_ref, g_ref, bt_ref, out_ref):
    y2 = y2_ref[...]
    z = (y2[:N] + y2[N:]) * dinv_ref[...] + b2_ref[...]
    mean = jnp.mean(z, axis=0, keepdims=True)
    zc = z - mean
    var = jnp.mean(zc * zc, axis=0, keepdims=True)
    out_ref[...] = zc * lax.rsqrt(var + 1e-5) * g_ref[...] + bt_ref[...]


def _bn(y2, dinv, b2, gamma, beta):
    return pl.pallas_call(
        _bn_body,
        out_shape=jax.ShapeDtypeStruct((N, D), jnp.float32),
    )(y2, dinv, b2.reshape(1, D), gamma.reshape(1, D), beta.reshape(1, D))


def kernel(edge_index, emb, W1, b1, W2, b2, gamma, beta):
    ei = edge_index.astype(jnp.int32)
    src, dst = ei[0], ei[1]
    pad = EP - E
    srcp = jnp.concatenate([src, jnp.zeros((pad,), jnp.int32)])
    dstp = jnp.concatenate([dst, jnp.full((pad,), N, jnp.int32)])
    src2 = jnp.concatenate([srcp, srcp + N])
    dst2 = jnp.concatenate([dstp, dstp])
    eh = E // NC
    ept = eh // NS
    ptile = EPT2 - ept
    src3h = jnp.pad(src.reshape(NC, NS, ept), ((0, 0), (0, 0), (0, ptile)))
    src3 = jnp.concatenate([src3h[0].reshape(-1), src3h[1].reshape(-1) + N])
    dst3 = jnp.pad(dst.reshape(NC * NS, ept), ((0, 0), (0, ptile)),
                   constant_values=N).reshape(-1)

    xw = _xw(emb, W1)
    deg2 = _deg_kernel(dst3, jnp.zeros((CH, D), jnp.float32),
                       jnp.ones((CH, D), jnp.float32))
    dinv, xs1 = _mm1(deg2, xw)
    y1 = _edge_scatter_l1(xs1, src2, dst2)
    xs2 = _mm2(y1, dinv, b1, W2)
    xs2p = jnp.concatenate([xs2, xs2, jnp.zeros((2 * N, D), jnp.float32)])
    y2 = _edge_scatter_l2(xs2p, src3, dst3)
    return _bn(y2, dinv, b2, gamma, beta)


# speedup vs baseline: 13.1959x; 1.0654x over previous
import functools

import jax
import jax.numpy as jnp
from jax import lax
from jax.experimental import pallas as pl
from jax.experimental.pallas import tpu as pltpu
from jax.experimental.pallas import tpu_sc as plsc

N = 10000
D = 128
E = 320000

NC = 2
NS = 16
CH = 128
NCHUNK = 158
EPT = CH * NCHUNK
EP = EPT * NS
NCHUNK2 = 80
EPT2 = CH * NCHUNK2
NP = N + 16

_MESH = plsc.VectorSubcoreMesh(core_axis_name="c", subcore_axis_name="s")


def _per_tile_rows(sid, total, fn):
    base = -(-((total + NS - 1) // NS) // 8) * 8
    last = total - (NS - 1) * base
    row0 = sid * base

    @pl.when(sid < NS - 1)
    def _():
        fn(row0, base)

    @pl.when(sid == NS - 1)
    def _():
        fn(row0, last)


def _deg_body(dst_hbm, zeros_hbm, ones_hbm, out_hbm, d_a, d_b, ones_v,
              stage_v, acc_s, sem_a, sem_b):
    cid = lax.axis_index("c")
    sid = lax.axis_index("s")
    pltpu.sync_copy(zeros_hbm, stage_v)
    pltpu.sync_copy(ones_hbm, ones_v)

    def zero(row0, sz):
        for off in range(0, sz, CH):
            c = min(CH, sz - off)
            pltpu.sync_copy(stage_v.at[pl.ds(0, c)],
                            acc_s.at[pl.ds(row0 + off, c)])

    _per_tile_rows(sid, NP, zero)
    plsc.subcore_barrier()
    ibase = (cid * NS + sid) * NCHUNK2 * CH

    def start_l(i, d, sem):
        pltpu.make_async_copy(dst_hbm.at[pl.ds(ibase + i * CH, CH)], d,
                              sem).start()

    def wait_l(i, d, sem):
        pltpu.make_async_copy(dst_hbm.at[pl.ds(ibase + i * CH, CH)], d,
                              sem).wait()

    P2 = NCHUNK2 // 2
    start_l(0, d_a, sem_a)

    def pair(g, _):
        start_l(2 * g + 1, d_b, sem_b)
        wait_l(2 * g, d_a, sem_a)
        pltpu.sync_copy(ones_v, acc_s.at[d_a], add=True)
        start_l(2 * g + 2, d_a, sem_a)
        wait_l(2 * g + 1, d_b, sem_b)
        pltpu.sync_copy(ones_v, acc_s.at[d_b], add=True)
        return 0

    lax.fori_loop(0, P2 - 1, pair, 0)
    start_l(NCHUNK2 - 1, d_b, sem_b)
    wait_l(NCHUNK2 - 2, d_a, sem_a)
    pltpu.sync_copy(ones_v, acc_s.at[d_a], add=True)
    wait_l(NCHUNK2 - 1, d_b, sem_b)
    pltpu.sync_copy(ones_v, acc_s.at[d_b], add=True)
    plsc.subcore_barrier()

    def writeout(row0, sz):
        pltpu.sync_copy(acc_s.at[pl.ds(row0, sz)],
                        out_hbm.at[pl.ds(cid * N + row0, sz)])

    _per_tile_rows(sid, N, writeout)


_deg_kernel = functools.partial(
    pl.kernel,
    out_type=jax.ShapeDtypeStruct((2 * N, D), jnp.float32),
    mesh=_MESH,
    scratch_types=[
        pltpu.VMEM((CH,), jnp.int32),
        pltpu.VMEM((CH,), jnp.int32),
        pltpu.VMEM((CH, D), jnp.float32),
        pltpu.VMEM((CH, D), jnp.float32),
        pltpu.VMEM_SHARED((NP, D), jnp.float32),
        pltpu.SemaphoreType.DMA,
        pltpu.SemaphoreType.DMA,
    ],
)(_deg_body)


def _make_edge_scatter(NCH):
    P = NCH // 2
    assert NCH == 2 * P

    def body(xs_hbm, src_hbm, dst_hbm, y_hbm, s_a, d_a, s_b, d_b,
             rows_a, rows_b, acc_s, sem_a, sem_b, sem_ls_a, sem_ld_a,
             sem_ls_b, sem_ld_b):
        cid = lax.axis_index("c")
        sid = lax.axis_index("s")
        nreg = xs_hbm.shape[0] // (2 * N)

        def seed(row0, sz):
            sbase = cid * N * (2 if nreg > 1 else 1)
            pltpu.sync_copy(xs_hbm.at[pl.ds(sbase + row0, sz)],
                            acc_s.at[pl.ds(row0, sz)])

        _per_tile_rows(sid, N, seed)
        plsc.subcore_barrier()

        ibase = (cid * NS + sid) * NCH * CH

        def start_l(i, s, d, ss, sd):
            pltpu.make_async_copy(src_hbm.at[pl.ds(ibase + i * CH, CH)], s,
                                  ss).start()
            pltpu.make_async_copy(dst_hbm.at[pl.ds(ibase + i * CH, CH)], d,
                                  sd).start()

        def wait_l(s, d, ss, sd):
            pltpu.make_async_copy(src_hbm.at[pl.ds(ibase, CH)], s, ss).wait()
            pltpu.make_async_copy(dst_hbm.at[pl.ds(ibase, CH)], d, sd).wait()

        def wait_la():
            wait_l(s_a, d_a, sem_ls_a, sem_ld_a)

        def wait_lb():
            wait_l(s_b, d_b, sem_ls_b, sem_ld_b)

        def start_a():
            pltpu.make_async_copy(xs_hbm.at[s_a], rows_a, sem_a).start()

        def wait_a():
            pltpu.make_async_copy(xs_hbm.at[s_a], rows_a, sem_a).wait()

        def start_b():
            pltpu.make_async_copy(xs_hbm.at[s_b], rows_b, sem_b).start()

        def wait_b():
            pltpu.make_async_copy(xs_hbm.at[s_b], rows_b, sem_b).wait()

        def scat_a():
            pltpu.sync_copy(rows_a, acc_s.at[d_a], add=True)

        def scat_b():
            pltpu.sync_copy(rows_b, acc_s.at[d_b], add=True)

        start_l(0, s_a, d_a, sem_ls_a, sem_ld_a)
        wait_la()
        start_a()
        start_l(1, s_b, d_b, sem_ls_b, sem_ld_b)

        def pair(g, _):
            wait_lb()
            wait_a()
            start_b()
            scat_a()
            start_l(2 * g + 2, s_a, d_a, sem_ls_a, sem_ld_a)
            wait_b()
            wait_la()
            start_a()
            scat_b()
            start_l(2 * g + 3, s_b, d_b, sem_ls_b, sem_ld_b)
            return 0

        lax.fori_loop(0, P - 1, pair, 0)
        wait_lb()
        wait_a()
        start_b()
        scat_a()
        wait_b()
        scat_b()
        plsc.subcore_barrier()

        def writeout(row0, sz):
            pltpu.sync_copy(acc_s.at[pl.ds(row0, sz)],
                            y_hbm.at[pl.ds(cid * N + row0, sz)])

        _per_tile_rows(sid, N, writeout)

    def make(xs, srcf, dstf):
        return functools.partial(
            pl.kernel,
            out_type=jax.ShapeDtypeStruct((2 * N, D), jnp.float32),
            mesh=_MESH,
            scratch_types=[
                pltpu.VMEM((CH,), jnp.int32),
                pltpu.VMEM((CH,), jnp.int32),
                pltpu.VMEM((CH,), jnp.int32),
                pltpu.VMEM((CH,), jnp.int32),
                pltpu.VMEM((CH, D), jnp.float32),
                pltpu.VMEM((CH, D), jnp.float32),
                pltpu.VMEM_SHARED((NP, D), jnp.float32),
                pltpu.SemaphoreType.DMA,
                pltpu.SemaphoreType.DMA,
                pltpu.SemaphoreType.DMA,
                pltpu.SemaphoreType.DMA,
                pltpu.SemaphoreType.DMA,
                pltpu.SemaphoreType.DMA,
            ],
        )(body)(xs, srcf, dstf)

    return make


_edge_scatter_l1 = _make_edge_scatter(NCHUNK)
_edge_scatter_l2 = _make_edge_scatter(NCHUNK2)

_BN = 1000
_GB = N // _BN


def _xw_body(emb_ref, w1_ref, xw_ref):
    xw_ref[...] = jnp.dot(emb_ref[...], w1_ref[...],
                          preferred_element_type=jnp.float32)


def _xw(emb, W1):
    return pl.pallas_call(
        _xw_body,
        grid=(_GB, 2),
        in_specs=[
            pl.BlockSpec((_BN, D), lambda i, j: (i, 0)),
            pl.BlockSpec((D, D), lambda i, j: (0, j)),
        ],
        out_specs=pl.BlockSpec((_BN, D), lambda i, j: (j * _GB + i, 0)),
        out_shape=jax.ShapeDtypeStruct((2 * N, D), jnp.float32),
    )(emb, W1)


def _scale_body(dega_ref, degb_ref, xw_ref, dinv_ref, xs_ref):
    deg = 1.0 + dega_ref[...][:, :1] + degb_ref[...][:, :1]
    dinv = lax.rsqrt(deg)
    dinv_ref[...] = dinv
    xs_ref[...] = xw_ref[...] * dinv


def _mm1(deg2, xw):
    return pl.pallas_call(
        _scale_body,
        grid=(_GB, 2),
        in_specs=[
            pl.BlockSpec((_BN, D), lambda i, j: (i, 0)),
            pl.BlockSpec((_BN, D), lambda i, j: (i + _GB, 0)),
            pl.BlockSpec((_BN, D), lambda i, j: (j * _GB + i, 0)),
        ],
        out_specs=[
            pl.BlockSpec((_BN, 1), lambda i, j: (i, 0)),
            pl.BlockSpec((_BN, D), lambda i, j: (j * _GB + i, 0)),
        ],
        out_shape=[
            jax.ShapeDtypeStruct((N, 1), jnp.float32),
            jax.ShapeDtypeStruct((2 * N, D), jnp.float32),
        ],
    )(deg2, deg2, xw)


def _mm2_body(y1a_ref, y1b_ref, dinv_ref, b1_ref, w2_ref, xs2_ref):
    dinv = dinv_ref[...]
    b1 = b1_ref[...]
    ha = y1a_ref[...] * dinv + b1[:, :D]
    hb = y1b_ref[...] * dinv + b1[:, D:]
    w2 = w2_ref[...]
    xw = (jnp.dot(ha, w2[:D], preferred_element_type=jnp.float32)
          + jnp.dot(hb, w2[D:], preferred_element_type=jnp.float32))
    xs2_ref[...] = xw * dinv


def _mm2(y1, dinv, b1, W2):
    return pl.pallas_call(
        _mm2_body,
        grid=(_GB,),
        in_specs=[
            pl.BlockSpec((_BN, D), lambda i: (i, 0)),
            pl.BlockSpec((_BN, D), lambda i: (i + _GB, 0)),
            pl.BlockSpec((_BN, 1), lambda i: (i, 0)),
            pl.BlockSpec((1, 2 * D), lambda i: (0, 0)),
            pl.BlockSpec((2 * D, D), lambda i: (0, 0)),
        ],
        out_specs=pl.BlockSpec((_BN, D), lambda i: (i, 0)),
        out_shape=jax.ShapeDtypeStruct((N, D), jnp.float32),
    )(y1, y1, dinv, b1.reshape(1, 2 * D), W2)


def _bn_body(y2_ref, dinv_ref, b2_ref, g_ref, bt_ref, out_ref):
    y2 = y2_ref[...]
    z = (y2[:N] + y2[N:]) * dinv_ref[...] + b2_ref[...]
    mean = jnp.mean(z, axis=0, keepdims=True)
    zc = z - mean
    var = jnp.mean(zc * zc, axis=0, keepdims=True)
    out_ref[...] = zc * lax.rsqrt(var + 1e-5) * g_ref[...] + bt_ref[...]


def _bn(y2, dinv, b2, gamma, beta):
    return pl.pallas_call(
        _bn_body,
        out_shape=jax.ShapeDtypeStruct((N, D), jnp.float32),
    )(y2, dinv, b2.reshape(1, D), gamma.reshape(1, D), beta.reshape(1, D))


def kernel(edge_index, emb, W1, b1, W2, b2, gamma, beta):
    ei = edge_index.astype(jnp.int32)
    src, dst = ei[0], ei[1]
    pad = EP - E
    srcp = jnp.concatenate([src, jnp.zeros((pad,), jnp.int32)])
    dstp = jnp.concatenate([dst, jnp.full((pad,), N, jnp.int32)])
    src2 = jnp.concatenate([srcp, srcp + N])
    dst2 = jnp.concatenate([dstp, dstp])
    eh = E // NC
    ept = eh // NS
    ptile = EPT2 - ept
    src3h = jnp.pad(src.reshape(NC, NS, ept), ((0, 0), (0, 0), (0, ptile)))
    src3 = jnp.concatenate([src3h[0].reshape(-1), src3h[1].reshape(-1) + N])
    dst3 = jnp.pad(dst.reshape(NC * NS, ept), ((0, 0), (0, ptile)),
                   constant_values=N).reshape(-1)

    xw = _xw(emb, W1)
    deg2 = _deg_kernel(dst3, jnp.zeros((CH, D), jnp.float32),
                       jnp.ones((CH, D), jnp.float32))
    dinv, xs1 = _mm1(deg2, xw)
    y1 = _edge_scatter_l1(xs1, src2, dst2)
    xs2 = _mm2(y1, dinv, b1, W2)
    xs2p = jnp.concatenate([xs2, xs2, jnp.zeros((2 * N, D), jnp.float32)])
    y2 = _edge_scatter_l2(xs2p, src3, dst3)
    return _bn(y2, dinv, b2, gamma, beta)
